# baseline (speedup 1.0000x reference)
import functools

import jax
import jax.numpy as jnp
from jax import lax
from jax.experimental import pallas as pl
from jax.experimental.pallas import tpu as pltpu
from jax.experimental.pallas import tpu_sc as plsc

D = 64
L = 16
BBLK = 128
NBUF = 3


def _rsqrt(x):
    i = plsc.bitcast(x, jnp.int32)
    i = jnp.int32(0x5F3759DF) - (i >> 1)
    y = plsc.bitcast(i, jnp.float32)
    h = x * jnp.float32(0.5)
    for _ in range(2):
        y = y * (jnp.float32(1.5) - h * y * y)
    return y


@functools.partial(jax.jit, static_argnames=("hist", "batch", "rows"))
def _embed_normalize(xt, ttf, hist, batch, rows):
    info = plsc.get_sparse_core_info()
    nc, ns = info.num_cores, info.num_subcores
    mesh = plsc.VectorSubcoreMesh(core_axis_name="c", subcore_axis_name="s")

    @functools.partial(
        pl.kernel,
        mesh=mesh,
        out_type=jax.ShapeDtypeStruct((hist, D, batch), jnp.float32),
        compiler_params=pltpu.CompilerParams(needs_layout_passes=False),
        scratch_types=[
            pltpu.VMEM((hist, BBLK), jnp.int32),
            pltpu.VMEM((NBUF, D, BBLK), jnp.float32),
            pltpu.SemaphoreType.DMA((NBUF,)),
            pltpu.SemaphoreType.DMA((NBUF,)),
        ],
    )
    def body(x_hbm, table_hbm, out_hbm, xb_v, gb_v, sem_g, sem_o):
        wid = lax.axis_index("s") * nc + lax.axis_index("c")
        bcol = wid * BBLK

        def slot(g):
            return lax.rem(g, NBUF)

        def gather_dma(g, d):
            b = slot(g)
            return pltpu.make_async_copy(
                table_hbm.at[pl.ds(d * rows, rows)].at[xb_v.at[g]],
                gb_v.at[b, d], sem_g.at[b])

        def start_gathers(g):
            for d in range(D):
                gather_dma(g, d).start()

        def wait_gathers(g):
            for d in range(D):
                gather_dma(g, d).wait()

        def out_dma(g):
            b = slot(g)
            return pltpu.make_async_copy(
                gb_v.at[b],
                out_hbm.at[g, :, pl.ds(bcol, BBLK)], sem_o.at[b])

        pltpu.sync_copy(x_hbm.at[:, pl.ds(bcol, BBLK)], xb_v)
        start_gathers(0)

        def chunk_body(g, carry):
            b = slot(g)

            @pl.when(g + 1 < hist)
            def _():
                @pl.when(g + 1 >= NBUF)
                def _():
                    out_dma(g + 1 - NBUF).wait()
                start_gathers(g + 1)

            wait_gathers(g)
            gb = gb_v.at[b]

            def grp(t, c):
                sl = pl.ds(t * L, L)
                ss = jnp.zeros((L,), jnp.float32)
                for d in range(D):
                    v = gb[d, sl]
                    ss = ss + v * v
                scale = _rsqrt(ss)
                for d in range(D):
                    gb[d, sl] = gb[d, sl] * scale
                return c

            lax.fori_loop(0, BBLK // L, grp, 0)
            out_dma(g).start()
            return carry

        lax.fori_loop(0, hist, chunk_body, 0)
        for t in range(NBUF):
            out_dma(hist - 1 - t).wait()

    return body(xt, ttf)


def kernel(x, table):
    b, h = x.shape
    out = _embed_normalize(x.T, table.T.reshape(-1), h, b, table.shape[0])
    return out.transpose(2, 0, 1)


# speedup vs baseline: 7.1347x; 7.1347x over previous
import functools

import jax
import jax.numpy as jnp
from jax import lax
from jax.experimental import pallas as pl
from jax.experimental.pallas import tpu as pltpu
from jax.experimental.pallas import tpu_sc as plsc

D = 64
L = 16
BBLK = 128
NBUF = 3


def _rsqrt(x):
    i = plsc.bitcast(x, jnp.int32)
    i = jnp.int32(0x5F3759DF) - (i >> 1)
    y = plsc.bitcast(i, jnp.float32)
    h = x * jnp.float32(0.5)
    for _ in range(2):
        y = y * (jnp.float32(1.5) - h * y * y)
    return y


def _rot_bases(lanes):
    return [(lanes + m) & (L - 1) for m in range(L)]


@functools.partial(jax.jit, static_argnames=("hist", "batch"))
def _embed_normalize(xt, table_p, hist, batch):
    info = plsc.get_sparse_core_info()
    nc, ns = info.num_cores, info.num_subcores
    mesh = plsc.VectorSubcoreMesh(core_axis_name="c", subcore_axis_name="s")

    @functools.partial(
        pl.kernel,
        mesh=mesh,
        out_type=jax.ShapeDtypeStruct((hist, D, batch), jnp.float32),
        compiler_params=pltpu.CompilerParams(needs_layout_passes=False),
        scratch_types=[
            pltpu.VMEM((hist, BBLK), jnp.int32),
            pltpu.VMEM((NBUF, BBLK, 2 * D), jnp.float32),
            pltpu.VMEM((NBUF, D, BBLK), jnp.float32),
            pltpu.SemaphoreType.DMA((NBUF,)),
            pltpu.SemaphoreType.DMA((NBUF,)),
        ],
    )
    def body(x_hbm, table_hbm, out_hbm, xb_v, gb_v, ob_v, sem_g, sem_o):
        wid = lax.axis_index("s") * nc + lax.axis_index("c")
        bcol = wid * BBLK

        def slot(g):
            return lax.rem(g, NBUF)

        def gather_dma(g):
            b = slot(g)
            return pltpu.make_async_copy(
                table_hbm.at[xb_v.at[g]], gb_v.at[b], sem_g.at[b])

        def out_dma(g):
            b = slot(g)
            return pltpu.make_async_copy(
                ob_v.at[b],
                out_hbm.at[g, :, pl.ds(bcol, BBLK)], sem_o.at[b])

        pltpu.sync_copy(x_hbm.at[:, pl.ds(bcol, BBLK)], xb_v)
        gather_dma(0).start()

        lanes = lax.iota(jnp.int32, L)
        rb = _rot_bases(lanes)

        def chunk_body(g, carry):
            b = slot(g)

            @pl.when(g + 1 < hist)
            def _():
                gather_dma(g + 1).start()

            gather_dma(g).wait()

            @pl.when(g >= NBUF)
            def _():
                out_dma(g - NBUF).wait()

            gb = gb_v.at[b]
            ob = ob_v.at[b]

            def grp(t, c):
                rowv = t * L + lanes
                ss = jnp.zeros((L,), jnp.float32)
                for d in range(D):
                    rv = rb[d & (L - 1)] + (d & ~(L - 1))
                    v = plsc.load_gather(gb, [rowv, rv])
                    ss = ss + v * v
                scale = _rsqrt(ss)
                for d in range(D):
                    rv = rb[d & (L - 1)] + (d & ~(L - 1))
                    v = plsc.load_gather(gb, [rowv, rv])
                    plsc.store_scatter(ob, [rv, rowv], v * scale)
                return c

            lax.fori_loop(0, BBLK // L, grp, 0)
            out_dma(g).start()
            return carry

        lax.fori_loop(0, hist, chunk_body, 0)
        for t in range(NBUF):
            out_dma(hist - 1 - t).wait()

    return body(xt, table_p)


def kernel(x, table):
    b, h = x.shape
    table_p = jnp.pad(table, ((0, 0), (0, table.shape[1])))
    out = _embed_normalize(x.T, table_p, h, b)
    return out.transpose(2, 0, 1)


# speedup vs baseline: 7.5048x; 1.0519x over previous
import functools

import jax
import jax.numpy as jnp
from jax import lax
from jax.experimental import pallas as pl
from jax.experimental.pallas import tpu as pltpu
from jax.experimental.pallas import tpu_sc as plsc

D = 64
L = 16
BBLK = 128
NBUF = 3


def _rsqrt(x):
    i = plsc.bitcast(x, jnp.int32)
    i = jnp.int32(0x5F3759DF) - (i >> 1)
    y = plsc.bitcast(i, jnp.float32)
    h = x * jnp.float32(0.5)
    for _ in range(2):
        y = y * (jnp.float32(1.5) - h * y * y)
    return y


@functools.partial(jax.jit, static_argnames=("hist", "batch"))
def _embed_normalize(xt, table_p, hist, batch):
    info = plsc.get_sparse_core_info()
    nc, ns = info.num_cores, info.num_subcores
    mesh = plsc.VectorSubcoreMesh(core_axis_name="c", subcore_axis_name="s")

    @functools.partial(
        pl.kernel,
        mesh=mesh,
        out_type=jax.ShapeDtypeStruct((hist, D, batch), jnp.float32),
        compiler_params=pltpu.CompilerParams(needs_layout_passes=False),
        scratch_types=[
            pltpu.VMEM((hist, BBLK), jnp.int32),
            pltpu.VMEM((NBUF, BBLK, 2 * D), jnp.float32),
            pltpu.VMEM((NBUF, D, BBLK), jnp.float32),
            pltpu.SemaphoreType.DMA((NBUF,)),
            pltpu.SemaphoreType.DMA((NBUF,)),
        ],
    )
    def body(x_hbm, table_hbm, out_hbm, xb_v, gb_v, ob_v, sem_g, sem_o):
        wid = lax.axis_index("s") * nc + lax.axis_index("c")
        bcol = wid * BBLK

        def slot(g):
            return lax.rem(g, NBUF)

        def gather_dma(g):
            b = slot(g)
            return pltpu.make_async_copy(
                table_hbm.at[xb_v.at[g]], gb_v.at[b], sem_g.at[b])

        def out_dma(g):
            b = slot(g)
            return pltpu.make_async_copy(
                ob_v.at[b],
                out_hbm.at[g, :, pl.ds(bcol, BBLK)], sem_o.at[b])

        pltpu.sync_copy(x_hbm.at[:, pl.ds(bcol, BBLK)], xb_v)
        gather_dma(0).start()

        lanes = lax.iota(jnp.int32, L)

        def chunk_body(g, carry):
            b = slot(g)

            @pl.when(g + 1 < hist)
            def _():
                gather_dma(g + 1).start()

            gather_dma(g).wait()

            @pl.when(g >= NBUF)
            def _():
                out_dma(g - NBUF).wait()

            gb = gb_v.at[b]
            ob = ob_v.at[b]

            def grp(t, c):
                rowv = t * L + lanes
                ss = jnp.zeros((L,), jnp.float32)
                rv = lanes
                for d in range(D):
                    v = plsc.load_gather(gb, [rowv, rv])
                    ss = ss + v * v
                    rv = rv + 1
                    rv = jnp.where(rv >= D, rv - D, rv)
                scale = _rsqrt(ss)
                rv = lanes
                for d in range(D):
                    v = plsc.load_gather(gb, [rowv, rv])
                    plsc.store_scatter(ob, [rv, rowv], v * scale)
                    rv = rv + 1
                    rv = jnp.where(rv >= D, rv - D, rv)
                return c

            lax.fori_loop(0, BBLK // L, grp, 0)
            out_dma(g).start()
            return carry

        lax.fori_loop(0, hist, chunk_body, 0)
        for t in range(NBUF):
            out_dma(hist - 1 - t).wait()

    return body(xt, table_p)


def kernel(x, table):
    b, h = x.shape
    table_p = jnp.pad(table, ((0, 0), (0, table.shape[1])))
    out = _embed_normalize(x.T, table_p, h, b)
    return out.transpose(2, 0, 1)
